# baseline (speedup 1.0000x reference)
import functools

import jax
import jax.numpy as jnp
from jax import lax
from jax.experimental import pallas as pl
from jax.experimental.pallas import tpu as pltpu
from jax.experimental.pallas import tpu_sc as plsc

N_NODES = 10000
N_EDGES = 160000
D_FEAT = 256
N_CLASSES = 40

NC = 2
NS = 16
CH = 128
CPP = 40
PH = 2
E_PAD = NS * PH * CPP * CH
N_TRASH = 16
ACC_ROWS = N_NODES + N_TRASH
NBUF = 2
ROWS_PER_TILE = 624
ZR = 16

_mesh = plsc.VectorSubcoreMesh(core_axis_name="c", subcore_axis_name="s")


def _fill_zero(ref, rows, width):
  zcols = width // 16
  def row(i, _):
    for k in range(zcols):
      ref[i, pl.ds(k * 16, 16)] = jnp.zeros((16,), jnp.float32)
    return 0
  lax.fori_loop(0, rows, row, 0)


def _zero_my_rows(acc_sp, zrow_v, s):
  base = s * ROWS_PER_TILE
  def blk(k, _):
    pltpu.sync_copy(zrow_v, acc_sp.at[pl.ds(base + k * ZR, ZR)])
    return 0
  lax.fori_loop(0, ROWS_PER_TILE // ZR, blk, 0)

  @pl.when(s == NS - 1)
  def _():
    pltpu.sync_copy(zrow_v, acc_sp.at[pl.ds(NS * ROWS_PER_TILE, ZR)])
    pltpu.sync_copy(zrow_v, acc_sp.at[pl.ds(N_NODES, N_TRASH)])


def _copy_my_rows(acc_sp, out_hbm, s):
  base = s * ROWS_PER_TILE
  pltpu.sync_copy(acc_sp.at[pl.ds(base, ROWS_PER_TILE)],
                  out_hbm.at[pl.ds(base, ROWS_PER_TILE)])

  @pl.when(s == NS - 1)
  def _():
    tail = NS * ROWS_PER_TILE
    pltpu.sync_copy(acc_sp.at[pl.ds(tail, N_NODES - tail)],
                    out_hbm.at[pl.ds(tail, N_NODES - tail)])


@functools.partial(
    pl.kernel,
    out_type=jax.ShapeDtypeStruct((NC, N_NODES, 16), jnp.float32),
    mesh=_mesh,
    scratch_types=[
        pltpu.VMEM((CPP, CH), jnp.int32),
        pltpu.VMEM((CH, 16), jnp.float32),
        pltpu.VMEM((ZR, 16), jnp.float32),
        pltpu.VMEM_SHARED((ACC_ROWS, 16), jnp.float32),
        pltpu.SemaphoreType.DMA,
    ],
)
def _deg_kernel(dst_hbm, degp_hbm, dstv, ones_v, zrow_v, deg_sp, sem):
  c = lax.axis_index("c")
  s = lax.axis_index("s")
  pltpu.sync_copy(dst_hbm.at[c, s], dstv)

  def fill_ones(i, _):
    ones_v[i, :] = jnp.ones((16,), jnp.float32)
    return 0
  lax.fori_loop(0, CH, fill_ones, 0)
  _fill_zero(zrow_v, ZR, 16)
  _zero_my_rows(deg_sp, zrow_v, s)
  plsc.subcore_barrier()

  def chunk(j, _):
    pltpu.make_async_copy(ones_v, deg_sp.at[dstv.at[j]], sem).start(add=True)
    return 0
  lax.fori_loop(0, CPP, chunk, 0)
  def drain(j, _):
    pltpu.make_async_copy(ones_v, deg_sp.at[dstv.at[j]], sem).wait()
    return 0
  lax.fori_loop(0, CPP, drain, 0)
  plsc.subcore_barrier()

  _copy_my_rows(deg_sp, degp_hbm.at[c], s)


def _pipelined_edges(g_hbm, srcv, dstv, rows_v, acc_sp, sem_g, sem_s, nchunk):
  for b in range(NBUF - 1):
    pltpu.make_async_copy(g_hbm.at[srcv.at[b]], rows_v.at[b], sem_g).start()

  def chunk(j, _):
    cur = lax.rem(j, NBUF)

    @pl.when(j >= 1)
    def _():
      prev = lax.rem(j - 1, NBUF)
      pltpu.make_async_copy(
          rows_v.at[prev], acc_sp.at[dstv.at[j - 1]], sem_s).wait()

    @pl.when(j + NBUF - 1 < nchunk)
    def _():
      nxt = lax.rem(j + NBUF - 1, NBUF)
      pltpu.make_async_copy(
          g_hbm.at[srcv.at[j + NBUF - 1]], rows_v.at[nxt], sem_g).start()

    pltpu.make_async_copy(g_hbm.at[srcv.at[j]], rows_v.at[cur], sem_g).wait()
    pltpu.make_async_copy(
        rows_v.at[cur], acc_sp.at[dstv.at[j]], sem_s).start(add=True)
    return 0

  lax.fori_loop(0, nchunk, chunk, 0)
  last = (nchunk - 1) % NBUF
  pltpu.make_async_copy(
      rows_v.at[last], acc_sp.at[dstv.at[nchunk - 1]], sem_s).wait()


def _make_edge_kernel(feat):
  out_nd = jax.ShapeDtypeStruct((N_NODES, feat), jnp.float32)

  @functools.partial(
      pl.kernel,
      out_type=(out_nd, out_nd),
      mesh=_mesh,
      scratch_types=[
          pltpu.VMEM((CPP, CH), jnp.int32),
          pltpu.VMEM((CPP, CH), jnp.int32),
          pltpu.VMEM((NBUF, CH, feat), jnp.float32),
          pltpu.VMEM((ZR, feat), jnp.float32),
          pltpu.VMEM_SHARED((ACC_ROWS, feat), jnp.float32),
          pltpu.SemaphoreType.DMA,
          pltpu.SemaphoreType.DMA,
      ],
  )
  def edge_kernel(glo_hbm, ghi_hbm, src_hbm, dst_hbm, slo_hbm, shi_hbm,
                  srcv, dstv, rows_v, zrow_v, acc_sp, sem_g, sem_s):
    c = lax.axis_index("c")
    s = lax.axis_index("s")

    _fill_zero(zrow_v, ZR, feat)
    _zero_my_rows(acc_sp, zrow_v, s)
    plsc.subcore_barrier()

    def run(g_hbm):
      for p in range(PH):
        pltpu.sync_copy(src_hbm.at[s, p], srcv)
        pltpu.sync_copy(dst_hbm.at[s, p], dstv)
        _pipelined_edges(g_hbm, srcv, dstv, rows_v, acc_sp, sem_g, sem_s, CPP)

    @pl.when(c == 0)
    def _():
      run(glo_hbm)

    @pl.when(c == 1)
    def _():
      run(ghi_hbm)

    plsc.subcore_barrier()

    @pl.when(c == 0)
    def _():
      _copy_my_rows(acc_sp, slo_hbm, s)

    @pl.when(c == 1)
    def _():
      _copy_my_rows(acc_sp, shi_hbm, s)

  return edge_kernel


_edge128 = _make_edge_kernel(128)


@functools.partial(
    pl.kernel,
    out_type=jax.ShapeDtypeStruct((NC, N_NODES, 128), jnp.float32),
    mesh=_mesh,
    scratch_types=[
        pltpu.VMEM((CPP, CH), jnp.int32),
        pltpu.VMEM((CPP, CH), jnp.int32),
        pltpu.VMEM((NBUF, CH, 128), jnp.float32),
        pltpu.VMEM((ZR, 128), jnp.float32),
        pltpu.VMEM_SHARED((ACC_ROWS, 128), jnp.float32),
        pltpu.SemaphoreType.DMA,
        pltpu.SemaphoreType.DMA,
    ],
)
def _edge_partial(g_hbm, src_hbm, dst_hbm, sp_hbm,
                  srcv, dstv, rows_v, zrow_v, acc_sp, sem_g, sem_s):
  c = lax.axis_index("c")
  s = lax.axis_index("s")
  pltpu.sync_copy(src_hbm.at[c, s], srcv)
  pltpu.sync_copy(dst_hbm.at[c, s], dstv)

  _fill_zero(zrow_v, ZR, 128)
  _zero_my_rows(acc_sp, zrow_v, s)
  plsc.subcore_barrier()

  _pipelined_edges(g_hbm, srcv, dstv, rows_v, acc_sp, sem_g, sem_s, CPP)
  plsc.subcore_barrier()

  _copy_my_rows(acc_sp, sp_hbm.at[c], s)


_BN = 1000


def _tc1_body(x_ref, w_ref, degp_ref, glo_ref, ghi_ref, dinv_ref):
  deg = 1.0 + degp_ref[0, :, 0:1] + degp_ref[1, :, 0:1]
  dinv = lax.rsqrt(deg)
  h = jnp.dot(x_ref[...], w_ref[...], preferred_element_type=jnp.float32)
  g = h * dinv
  glo_ref[...] = g[:, :128]
  ghi_ref[...] = g[:, 128:]
  dinv_ref[...] = dinv


def _tc1(x, w0, degp):
  grid = (N_NODES // _BN,)
  return pl.pallas_call(
      _tc1_body,
      grid=grid,
      in_specs=[
          pl.BlockSpec((_BN, D_FEAT), lambda i: (i, 0)),
          pl.BlockSpec((D_FEAT, D_FEAT), lambda i: (0, 0)),
          pl.BlockSpec((NC, _BN, 16), lambda i: (0, i, 0)),
      ],
      out_specs=[
          pl.BlockSpec((_BN, 128), lambda i: (i, 0)),
          pl.BlockSpec((_BN, 128), lambda i: (i, 0)),
          pl.BlockSpec((_BN, 1), lambda i: (i, 0)),
      ],
      out_shape=[
          jax.ShapeDtypeStruct((N_NODES, 128), jnp.float32),
          jax.ShapeDtypeStruct((N_NODES, 128), jnp.float32),
          jax.ShapeDtypeStruct((N_NODES, 1), jnp.float32),
      ],
  )(x, w0, degp)


def _tc_mid_body(slo_ref, shi_ref, glo_ref, ghi_ref, dinv_ref, w_ref,
                 olo_ref, ohi_ref=None, *, dout):
  dinv = dinv_ref[...]
  alo = jnp.maximum((slo_ref[...] + glo_ref[...]) * dinv, 0.0)
  ahi = jnp.maximum((shi_ref[...] + ghi_ref[...]) * dinv, 0.0)
  h = (jnp.dot(alo, w_ref[0:128, :], preferred_element_type=jnp.float32)
       + jnp.dot(ahi, w_ref[128:256, :], preferred_element_type=jnp.float32))
  g = h * dinv
  if dout == 256:
    olo_ref[...] = g[:, :128]
    ohi_ref[...] = g[:, 128:]
  else:
    olo_ref[...] = jnp.concatenate(
        [g, jnp.zeros((g.shape[0], 128 - dout), jnp.float32)], axis=1)


def _tc_mid(slo, shi, glo, ghi, dinv, w):
  dout = w.shape[1]
  grid = (N_NODES // _BN,)
  if dout == 256:
    out_specs = [pl.BlockSpec((_BN, 128), lambda i: (i, 0)),
                 pl.BlockSpec((_BN, 128), lambda i: (i, 0))]
    out_shape = [jax.ShapeDtypeStruct((N_NODES, 128), jnp.float32),
                 jax.ShapeDtypeStruct((N_NODES, 128), jnp.float32)]
  else:
    out_specs = [pl.BlockSpec((_BN, 128), lambda i: (i, 0))]
    out_shape = [jax.ShapeDtypeStruct((N_NODES, 128), jnp.float32)]
  return pl.pallas_call(
      functools.partial(_tc_mid_body, dout=dout),
      grid=grid,
      in_specs=[
          pl.BlockSpec((_BN, 128), lambda i: (i, 0)),
          pl.BlockSpec((_BN, 128), lambda i: (i, 0)),
          pl.BlockSpec((_BN, 128), lambda i: (i, 0)),
          pl.BlockSpec((_BN, 128), lambda i: (i, 0)),
          pl.BlockSpec((_BN, 1), lambda i: (i, 0)),
          pl.BlockSpec((D_FEAT, dout), lambda i: (0, 0)),
      ],
      out_specs=out_specs,
      out_shape=out_shape,
  )(slo, shi, glo, ghi, dinv, w)


def _tc_final_body(sp_ref, g_ref, dinv_ref, out_ref):
  z = sp_ref[0] + sp_ref[1] + g_ref[...]
  out_ref[...] = jnp.maximum(z * dinv_ref[...], 0.0)[:, :N_CLASSES]


def _tc_final(sp, g, dinv):
  grid = (N_NODES // _BN,)
  return pl.pallas_call(
      _tc_final_body,
      grid=grid,
      in_specs=[
          pl.BlockSpec((NC, _BN, 128), lambda i: (0, i, 0)),
          pl.BlockSpec((_BN, 128), lambda i: (i, 0)),
          pl.BlockSpec((_BN, 1), lambda i: (i, 0)),
      ],
      out_specs=pl.BlockSpec((_BN, N_CLASSES), lambda i: (i, 0)),
      out_shape=jax.ShapeDtypeStruct((N_NODES, N_CLASSES), jnp.float32),
  )(sp, g, dinv)


def kernel(x, edge_index, W0, W1, W2):
  src = edge_index[0].astype(jnp.int32)
  dst = edge_index[1].astype(jnp.int32)
  npad = E_PAD - N_EDGES
  src_p = jnp.concatenate([src, jnp.zeros((npad,), jnp.int32)])
  dst_p = jnp.concatenate(
      [dst, N_NODES + (jnp.arange(npad, dtype=jnp.int32) % N_TRASH)])
  src_f = src_p.reshape(NS, PH, CPP, CH)
  dst_f = dst_p.reshape(NS, PH, CPP, CH)
  src_e = src_p.reshape(NC, NS, CPP, CH)
  dst_e = dst_p.reshape(NC, NS, CPP, CH)

  degp = _deg_kernel(dst_e)
  glo0, ghi0, dinv = _tc1(x, W0, degp)
  slo0, shi0 = _edge128(glo0, ghi0, src_f, dst_f)
  glo1, ghi1 = _tc_mid(slo0, shi0, glo0, ghi0, dinv, W1)
  slo1, shi1 = _edge128(glo1, ghi1, src_f, dst_f)
  (g2,) = _tc_mid(slo1, shi1, glo1, ghi1, dinv, W2)
  s2p = _edge_partial(g2, src_e, dst_e)
  return _tc_final(s2p, g2, dinv)


# speedup vs baseline: 1.1251x; 1.1251x over previous
import functools

import jax
import jax.numpy as jnp
from jax import lax
from jax.experimental import pallas as pl
from jax.experimental.pallas import tpu as pltpu
from jax.experimental.pallas import tpu_sc as plsc

N_NODES = 10000
N_EDGES = 160000
D_FEAT = 256
N_CLASSES = 40

NC = 2
NS = 16
CH = 128
CPP = 40
PH = 2
N_TRASH = 128
ACC_ROWS = N_NODES + N_TRASH
NBUF = 2
ROWS_PER_TILE = 624
ZR = 16

_mesh = plsc.VectorSubcoreMesh(core_axis_name="c", subcore_axis_name="s")


def _fill_zero(ref, rows, width):
  zcols = width // 16
  def row(i, _):
    for k in range(zcols):
      ref[i, pl.ds(k * 16, 16)] = jnp.zeros((16,), jnp.float32)
    return 0
  lax.fori_loop(0, rows, row, 0)


def _zero_my_rows(acc_sp, zrow_v, s):
  base = s * ROWS_PER_TILE
  def blk(k, _):
    pltpu.sync_copy(zrow_v, acc_sp.at[pl.ds(base + k * ZR, ZR)])
    return 0
  lax.fori_loop(0, ROWS_PER_TILE // ZR, blk, 0)

  @pl.when(s == NS - 1)
  def _():
    def tailblk(k, _):
      pltpu.sync_copy(
          zrow_v, acc_sp.at[pl.ds(NS * ROWS_PER_TILE + k * ZR, ZR)])
      return 0
    lax.fori_loop(0, (ACC_ROWS - NS * ROWS_PER_TILE) // ZR, tailblk, 0)


def _copy_my_rows(acc_sp, out_hbm, s):
  base = s * ROWS_PER_TILE
  pltpu.sync_copy(acc_sp.at[pl.ds(base, ROWS_PER_TILE)],
                  out_hbm.at[pl.ds(base, ROWS_PER_TILE)])

  @pl.when(s == NS - 1)
  def _():
    tail = NS * ROWS_PER_TILE
    pltpu.sync_copy(acc_sp.at[pl.ds(tail, N_NODES - tail)],
                    out_hbm.at[pl.ds(tail, N_NODES - tail)])


@functools.partial(
    pl.kernel,
    out_type=jax.ShapeDtypeStruct((NC, N_NODES, 16), jnp.float32),
    mesh=_mesh,
    scratch_types=[
        pltpu.VMEM((CPP, CH), jnp.int32),
        pltpu.VMEM((CH, 16), jnp.float32),
        pltpu.VMEM((ZR, 16), jnp.float32),
        pltpu.VMEM_SHARED((ACC_ROWS, 16), jnp.float32),
        pltpu.SemaphoreType.DMA,
    ],
)
def _deg_kernel(dst_hbm, degp_hbm, dstv, ones_v, zrow_v, deg_sp, sem):
  c = lax.axis_index("c")
  s = lax.axis_index("s")
  pltpu.sync_copy(dst_hbm.at[c, s], dstv)

  def fill_ones(i, _):
    ones_v[i, :] = jnp.ones((16,), jnp.float32)
    return 0
  lax.fori_loop(0, CH, fill_ones, 0)
  _fill_zero(zrow_v, ZR, 16)
  _zero_my_rows(deg_sp, zrow_v, s)
  plsc.subcore_barrier()

  def chunk(j, _):
    pltpu.make_async_copy(ones_v, deg_sp.at[dstv.at[j]], sem).start(add=True)
    return 0
  lax.fori_loop(0, CPP, chunk, 0)
  def drain(j, _):
    pltpu.make_async_copy(ones_v, deg_sp.at[dstv.at[j]], sem).wait()
    return 0
  lax.fori_loop(0, CPP, drain, 0)
  plsc.subcore_barrier()

  _copy_my_rows(deg_sp, degp_hbm.at[c], s)


def _pipelined_edges(g_hbm, srcv, dstv, rows_v, acc_sp, sem_g, sem_s, nchunk):
  for b in range(NBUF - 1):
    pltpu.make_async_copy(g_hbm.at[srcv.at[b]], rows_v.at[b], sem_g).start()

  def chunk(j, _):
    cur = lax.rem(j, NBUF)

    @pl.when(j >= 1)
    def _():
      prev = lax.rem(j - 1, NBUF)
      pltpu.make_async_copy(
          rows_v.at[prev], acc_sp.at[dstv.at[j - 1]], sem_s).wait()

    @pl.when(j + NBUF - 1 < nchunk)
    def _():
      nxt = lax.rem(j + NBUF - 1, NBUF)
      pltpu.make_async_copy(
          g_hbm.at[srcv.at[j + NBUF - 1]], rows_v.at[nxt], sem_g).start()

    pltpu.make_async_copy(g_hbm.at[srcv.at[j]], rows_v.at[cur], sem_g).wait()
    pltpu.make_async_copy(
        rows_v.at[cur], acc_sp.at[dstv.at[j]], sem_s).start(add=True)
    return 0

  lax.fori_loop(0, nchunk, chunk, 0)
  last = (nchunk - 1) % NBUF
  pltpu.make_async_copy(
      rows_v.at[last], acc_sp.at[dstv.at[nchunk - 1]], sem_s).wait()


def _make_edge_kernel(feat):
  out_nd = jax.ShapeDtypeStruct((N_NODES, feat), jnp.float32)

  @functools.partial(
      pl.kernel,
      out_type=(out_nd, out_nd),
      mesh=_mesh,
      scratch_types=[
          pltpu.VMEM((CPP, CH), jnp.int32),
          pltpu.VMEM((CPP, CH), jnp.int32),
          pltpu.VMEM((NBUF, CH, feat), jnp.float32),
          pltpu.VMEM((ZR, feat), jnp.float32),
          pltpu.VMEM_SHARED((ACC_ROWS, feat), jnp.float32),
          pltpu.SemaphoreType.DMA,
          pltpu.SemaphoreType.DMA,
      ],
  )
  def edge_kernel(glo_hbm, ghi_hbm, src_hbm, dst_hbm, slo_hbm, shi_hbm,
                  srcv, dstv, rows_v, zrow_v, acc_sp, sem_g, sem_s):
    c = lax.axis_index("c")
    s = lax.axis_index("s")

    _fill_zero(zrow_v, ZR, feat)
    _zero_my_rows(acc_sp, zrow_v, s)
    plsc.subcore_barrier()

    def run(g_hbm):
      for p in range(PH):
        pltpu.sync_copy(src_hbm.at[s, p], srcv)
        pltpu.sync_copy(dst_hbm.at[s, p], dstv)
        _pipelined_edges(g_hbm, srcv, dstv, rows_v, acc_sp, sem_g, sem_s, CPP)

    @pl.when(c == 0)
    def _():
      run(glo_hbm)

    @pl.when(c == 1)
    def _():
      run(ghi_hbm)

    plsc.subcore_barrier()

    @pl.when(c == 0)
    def _():
      _copy_my_rows(acc_sp, slo_hbm, s)

    @pl.when(c == 1)
    def _():
      _copy_my_rows(acc_sp, shi_hbm, s)

  return edge_kernel


_edge128 = _make_edge_kernel(128)


@functools.partial(
    pl.kernel,
    out_type=jax.ShapeDtypeStruct((NC, N_NODES, 128), jnp.float32),
    mesh=_mesh,
    scratch_types=[
        pltpu.VMEM((CPP, CH), jnp.int32),
        pltpu.VMEM((CPP, CH), jnp.int32),
        pltpu.VMEM((NBUF, CH, 128), jnp.float32),
        pltpu.VMEM((ZR, 128), jnp.float32),
        pltpu.VMEM_SHARED((ACC_ROWS, 128), jnp.float32),
        pltpu.SemaphoreType.DMA,
        pltpu.SemaphoreType.DMA,
    ],
)
def _edge_partial(g_hbm, src_hbm, dst_hbm, sp_hbm,
                  srcv, dstv, rows_v, zrow_v, acc_sp, sem_g, sem_s):
  c = lax.axis_index("c")
  s = lax.axis_index("s")
  pltpu.sync_copy(src_hbm.at[c, s], srcv)
  pltpu.sync_copy(dst_hbm.at[c, s], dstv)

  _fill_zero(zrow_v, ZR, 128)
  _zero_my_rows(acc_sp, zrow_v, s)
  plsc.subcore_barrier()

  _pipelined_edges(g_hbm, srcv, dstv, rows_v, acc_sp, sem_g, sem_s, CPP)
  plsc.subcore_barrier()

  _copy_my_rows(acc_sp, sp_hbm.at[c], s)


_BN = 1000


def _tc1_body(x_ref, w_ref, degp_ref, glo_ref, ghi_ref, dinv_ref):
  deg = 1.0 + degp_ref[0, :, 0:1] + degp_ref[1, :, 0:1]
  dinv = lax.rsqrt(deg)
  h = jnp.dot(x_ref[...], w_ref[...], preferred_element_type=jnp.float32)
  g = h * dinv
  glo_ref[...] = g[:, :128]
  ghi_ref[...] = g[:, 128:]
  dinv_ref[...] = dinv


def _tc1(x, w0, degp):
  grid = (N_NODES // _BN,)
  return pl.pallas_call(
      _tc1_body,
      grid=grid,
      in_specs=[
          pl.BlockSpec((_BN, D_FEAT), lambda i: (i, 0)),
          pl.BlockSpec((D_FEAT, D_FEAT), lambda i: (0, 0)),
          pl.BlockSpec((NC, _BN, 16), lambda i: (0, i, 0)),
      ],
      out_specs=[
          pl.BlockSpec((_BN, 128), lambda i: (i, 0)),
          pl.BlockSpec((_BN, 128), lambda i: (i, 0)),
          pl.BlockSpec((_BN, 1), lambda i: (i, 0)),
      ],
      out_shape=[
          jax.ShapeDtypeStruct((N_NODES, 128), jnp.float32),
          jax.ShapeDtypeStruct((N_NODES, 128), jnp.float32),
          jax.ShapeDtypeStruct((N_NODES, 1), jnp.float32),
      ],
  )(x, w0, degp)


def _tc_mid_body(slo_ref, shi_ref, glo_ref, ghi_ref, dinv_ref, w_ref,
                 olo_ref, ohi_ref=None, *, dout):
  dinv = dinv_ref[...]
  alo = jnp.maximum((slo_ref[...] + glo_ref[...]) * dinv, 0.0)
  ahi = jnp.maximum((shi_ref[...] + ghi_ref[...]) * dinv, 0.0)
  h = (jnp.dot(alo, w_ref[0:128, :], preferred_element_type=jnp.float32)
       + jnp.dot(ahi, w_ref[128:256, :], preferred_element_type=jnp.float32))
  g = h * dinv
  if dout == 256:
    olo_ref[...] = g[:, :128]
    ohi_ref[...] = g[:, 128:]
  else:
    olo_ref[...] = jnp.concatenate(
        [g, jnp.zeros((g.shape[0], 128 - dout), jnp.float32)], axis=1)


def _tc_mid(slo, shi, glo, ghi, dinv, w):
  dout = w.shape[1]
  grid = (N_NODES // _BN,)
  if dout == 256:
    out_specs = [pl.BlockSpec((_BN, 128), lambda i: (i, 0)),
                 pl.BlockSpec((_BN, 128), lambda i: (i, 0))]
    out_shape = [jax.ShapeDtypeStruct((N_NODES, 128), jnp.float32),
                 jax.ShapeDtypeStruct((N_NODES, 128), jnp.float32)]
  else:
    out_specs = [pl.BlockSpec((_BN, 128), lambda i: (i, 0))]
    out_shape = [jax.ShapeDtypeStruct((N_NODES, 128), jnp.float32)]
  return pl.pallas_call(
      functools.partial(_tc_mid_body, dout=dout),
      grid=grid,
      in_specs=[
          pl.BlockSpec((_BN, 128), lambda i: (i, 0)),
          pl.BlockSpec((_BN, 128), lambda i: (i, 0)),
          pl.BlockSpec((_BN, 128), lambda i: (i, 0)),
          pl.BlockSpec((_BN, 128), lambda i: (i, 0)),
          pl.BlockSpec((_BN, 1), lambda i: (i, 0)),
          pl.BlockSpec((D_FEAT, dout), lambda i: (0, 0)),
      ],
      out_specs=out_specs,
      out_shape=out_shape,
  )(slo, shi, glo, ghi, dinv, w)


def _tc_final_body(sp_ref, g_ref, dinv_ref, out_ref):
  z = sp_ref[0] + sp_ref[1] + g_ref[...]
  out_ref[...] = jnp.maximum(z * dinv_ref[...], 0.0)[:, :N_CLASSES]


def _tc_final(sp, g, dinv):
  grid = (N_NODES // _BN,)
  return pl.pallas_call(
      _tc_final_body,
      grid=grid,
      in_specs=[
          pl.BlockSpec((NC, _BN, 128), lambda i: (0, i, 0)),
          pl.BlockSpec((_BN, 128), lambda i: (i, 0)),
          pl.BlockSpec((_BN, 1), lambda i: (i, 0)),
      ],
      out_specs=pl.BlockSpec((_BN, N_CLASSES), lambda i: (i, 0)),
      out_shape=jax.ShapeDtypeStruct((N_NODES, N_CLASSES), jnp.float32),
  )(sp, g, dinv)


def kernel(x, edge_index, W0, W1, W2):
  src = edge_index[0].astype(jnp.int32)
  dst = edge_index[1].astype(jnp.int32)
  def pad_split(v, trash, groups):
    per = N_EDGES // groups
    per_pad = PH * CPP * CH * NS // groups
    v2 = v.reshape(groups, per)
    if trash:
      fill = jnp.broadcast_to(
          N_NODES + (jnp.arange(per_pad - per, dtype=jnp.int32) % N_TRASH),
          (groups, per_pad - per))
    else:
      fill = jnp.zeros((groups, per_pad - per), jnp.int32)
    return jnp.concatenate([v2, fill], axis=1)

  src_f = pad_split(src, False, NS).reshape(NS, PH, CPP, CH)
  dst_f = pad_split(dst, True, NS).reshape(NS, PH, CPP, CH)
  src_e = pad_split(src, False, NC * NS).reshape(NC, NS, CPP, CH)
  dst_e = pad_split(dst, True, NC * NS).reshape(NC, NS, CPP, CH)

  degp = _deg_kernel(dst_e)
  glo0, ghi0, dinv = _tc1(x, W0, degp)
  slo0, shi0 = _edge128(glo0, ghi0, src_f, dst_f)
  glo1, ghi1 = _tc_mid(slo0, shi0, glo0, ghi0, dinv, W1)
  slo1, shi1 = _edge128(glo1, ghi1, src_f, dst_f)
  (g2,) = _tc_mid(slo1, shi1, glo1, ghi1, dinv, W2)
  s2p = _edge_partial(g2, src_e, dst_e)
  return _tc_final(s2p, g2, dinv)


# speedup vs baseline: 2.3453x; 2.0846x over previous
import functools

import jax
import jax.numpy as jnp
from jax import lax
from jax.experimental import pallas as pl
from jax.experimental.pallas import tpu as pltpu
from jax.experimental.pallas import tpu_sc as plsc

N_NODES = 10000
N_EDGES = 160000
D_FEAT = 256
N_CLASSES = 40

NC = 2
NS = 16
CH = 128
CPP = 40
PH = 2
N_TRASH = 128
ACC_ROWS = N_NODES + N_TRASH
NBUF = 2
ROWS_PER_TILE = 624
ZR = 16

_mesh = plsc.VectorSubcoreMesh(core_axis_name="c", subcore_axis_name="s")


def _fill_zero(ref, rows, width):
  zcols = width // 16
  def row(i, _):
    for k in range(zcols):
      ref[i, pl.ds(k * 16, 16)] = jnp.zeros((16,), jnp.float32)
    return 0
  lax.fori_loop(0, rows, row, 0)


def _zero_my_rows(acc_sp, zrow_v, s):
  base = s * ROWS_PER_TILE
  def blk(k, _):
    pltpu.sync_copy(zrow_v, acc_sp.at[pl.ds(base + k * ZR, ZR)])
    return 0
  lax.fori_loop(0, ROWS_PER_TILE // ZR, blk, 0)

  @pl.when(s == NS - 1)
  def _():
    def tailblk(k, _):
      pltpu.sync_copy(
          zrow_v, acc_sp.at[pl.ds(NS * ROWS_PER_TILE + k * ZR, ZR)])
      return 0
    lax.fori_loop(0, (ACC_ROWS - NS * ROWS_PER_TILE) // ZR, tailblk, 0)


def _copy_my_rows(acc_sp, out_hbm, s):
  base = s * ROWS_PER_TILE
  pltpu.sync_copy(acc_sp.at[pl.ds(base, ROWS_PER_TILE)],
                  out_hbm.at[pl.ds(base, ROWS_PER_TILE)])

  @pl.when(s == NS - 1)
  def _():
    tail = NS * ROWS_PER_TILE
    pltpu.sync_copy(acc_sp.at[pl.ds(tail, N_NODES - tail)],
                    out_hbm.at[pl.ds(tail, N_NODES - tail)])


@functools.partial(
    pl.kernel,
    out_type=jax.ShapeDtypeStruct((NC, N_NODES, 16), jnp.float32),
    mesh=_mesh,
    scratch_types=[
        pltpu.VMEM((CPP, CH), jnp.int32),
        pltpu.VMEM((CH, 16), jnp.float32),
        pltpu.VMEM((ZR, 16), jnp.float32),
        pltpu.VMEM_SHARED((ACC_ROWS, 16), jnp.float32),
        pltpu.SemaphoreType.DMA,
    ],
)
def _deg_kernel(dst_hbm, degp_hbm, dstv, ones_v, zrow_v, deg_sp, sem):
  c = lax.axis_index("c")
  s = lax.axis_index("s")
  pltpu.sync_copy(dst_hbm.at[c, s], dstv)

  def fill_ones(i, _):
    ones_v[i, :] = jnp.ones((16,), jnp.float32)
    return 0
  lax.fori_loop(0, CH, fill_ones, 0)
  _fill_zero(zrow_v, ZR, 16)
  _zero_my_rows(deg_sp, zrow_v, s)
  plsc.subcore_barrier()

  def chunk(j, _):
    pltpu.make_async_copy(ones_v, deg_sp.at[dstv.at[j]], sem).start(add=True)
    return 0
  lax.fori_loop(0, CPP, chunk, 0)
  def drain(j, _):
    pltpu.make_async_copy(ones_v, deg_sp.at[dstv.at[j]], sem).wait()
    return 0
  lax.fori_loop(0, CPP, drain, 0)
  plsc.subcore_barrier()

  _copy_my_rows(deg_sp, degp_hbm.at[c], s)


def _pipelined_edges(g_hbm, srcv, dstv, rows_v, acc_sp, sem_g, sem_s, nchunk):
  for b in range(NBUF - 1):
    pltpu.make_async_copy(g_hbm.at[srcv.at[b]], rows_v.at[b], sem_g).start()

  def chunk(j, _):
    cur = lax.rem(j, NBUF)

    @pl.when(j >= 1)
    def _():
      prev = lax.rem(j - 1, NBUF)
      pltpu.make_async_copy(
          rows_v.at[prev], acc_sp.at[dstv.at[j - 1]], sem_s).wait()

    @pl.when(j + NBUF - 1 < nchunk)
    def _():
      nxt = lax.rem(j + NBUF - 1, NBUF)
      pltpu.make_async_copy(
          g_hbm.at[srcv.at[j + NBUF - 1]], rows_v.at[nxt], sem_g).start()

    pltpu.make_async_copy(g_hbm.at[srcv.at[j]], rows_v.at[cur], sem_g).wait()
    pltpu.make_async_copy(
        rows_v.at[cur], acc_sp.at[dstv.at[j]], sem_s).start(add=True)
    return 0

  lax.fori_loop(0, nchunk, chunk, 0)
  last = (nchunk - 1) % NBUF
  pltpu.make_async_copy(
      rows_v.at[last], acc_sp.at[dstv.at[nchunk - 1]], sem_s).wait()


def _make_edge_kernel(feat):
  out_nd = jax.ShapeDtypeStruct((N_NODES, feat), jnp.float32)

  @functools.partial(
      pl.kernel,
      out_type=(out_nd, out_nd),
      mesh=_mesh,
      scratch_types=[
          pltpu.VMEM((CPP, CH), jnp.int32),
          pltpu.VMEM((CPP, CH), jnp.int32),
          pltpu.VMEM((NBUF, CH, feat), jnp.float32),
          pltpu.VMEM((ZR, feat), jnp.float32),
          pltpu.VMEM_SHARED((ACC_ROWS, feat), jnp.float32),
          pltpu.SemaphoreType.DMA,
          pltpu.SemaphoreType.DMA,
      ],
  )
  def edge_kernel(glo_hbm, ghi_hbm, src_hbm, dst_hbm, slo_hbm, shi_hbm,
                  srcv, dstv, rows_v, zrow_v, acc_sp, sem_g, sem_s):
    c = lax.axis_index("c")
    s = lax.axis_index("s")

    _fill_zero(zrow_v, ZR, feat)
    _zero_my_rows(acc_sp, zrow_v, s)
    plsc.subcore_barrier()

    def run(g_hbm):
      for p in range(PH):
        pltpu.sync_copy(src_hbm.at[s, p], srcv)
        pltpu.sync_copy(dst_hbm.at[s, p], dstv)
        _pipelined_edges(g_hbm, srcv, dstv, rows_v, acc_sp, sem_g, sem_s, CPP)

    @pl.when(c == 0)
    def _():
      run(glo_hbm)

    @pl.when(c == 1)
    def _():
      run(ghi_hbm)

    plsc.subcore_barrier()

    @pl.when(c == 0)
    def _():
      _copy_my_rows(acc_sp, slo_hbm, s)

    @pl.when(c == 1)
    def _():
      _copy_my_rows(acc_sp, shi_hbm, s)

  return edge_kernel


_edge128 = _make_edge_kernel(128)


@functools.partial(
    pl.kernel,
    out_type=jax.ShapeDtypeStruct((NC, N_NODES, 128), jnp.float32),
    mesh=_mesh,
    scratch_types=[
        pltpu.VMEM((CPP, CH), jnp.int32),
        pltpu.VMEM((CPP, CH), jnp.int32),
        pltpu.VMEM((NBUF, CH, 128), jnp.float32),
        pltpu.VMEM((ZR, 128), jnp.float32),
        pltpu.VMEM_SHARED((ACC_ROWS, 128), jnp.float32),
        pltpu.SemaphoreType.DMA,
        pltpu.SemaphoreType.DMA,
    ],
)
def _edge_partial(g_hbm, src_hbm, dst_hbm, sp_hbm,
                  srcv, dstv, rows_v, zrow_v, acc_sp, sem_g, sem_s):
  c = lax.axis_index("c")
  s = lax.axis_index("s")
  pltpu.sync_copy(src_hbm.at[c, s], srcv)
  pltpu.sync_copy(dst_hbm.at[c, s], dstv)

  _fill_zero(zrow_v, ZR, 128)
  _zero_my_rows(acc_sp, zrow_v, s)
  plsc.subcore_barrier()

  _pipelined_edges(g_hbm, srcv, dstv, rows_v, acc_sp, sem_g, sem_s, CPP)
  plsc.subcore_barrier()

  _copy_my_rows(acc_sp, sp_hbm.at[c], s)


_BN = 1000


def _tc1_body(x_ref, w_ref, degp_ref, glo_ref, ghi_ref, dinv_ref):
  deg = 1.0 + degp_ref[0, :, 0:1] + degp_ref[1, :, 0:1]
  dinv = lax.rsqrt(deg)
  h = jnp.dot(x_ref[...], w_ref[...], preferred_element_type=jnp.float32)
  g = h * dinv
  glo_ref[...] = g[:, :128]
  ghi_ref[...] = g[:, 128:]
  dinv_ref[...] = dinv


def _tc1(x, w0, degp):
  grid = (N_NODES // _BN,)
  return pl.pallas_call(
      _tc1_body,
      grid=grid,
      in_specs=[
          pl.BlockSpec((_BN, D_FEAT), lambda i: (i, 0)),
          pl.BlockSpec((D_FEAT, D_FEAT), lambda i: (0, 0)),
          pl.BlockSpec((NC, _BN, 16), lambda i: (0, i, 0)),
      ],
      out_specs=[
          pl.BlockSpec((_BN, 128), lambda i: (i, 0)),
          pl.BlockSpec((_BN, 128), lambda i: (i, 0)),
          pl.BlockSpec((_BN, 1), lambda i: (i, 0)),
      ],
      out_shape=[
          jax.ShapeDtypeStruct((N_NODES, 128), jnp.float32),
          jax.ShapeDtypeStruct((N_NODES, 128), jnp.float32),
          jax.ShapeDtypeStruct((N_NODES, 1), jnp.float32),
      ],
  )(x, w0, degp)


def _tc_mid_body(slo_ref, shi_ref, glo_ref, ghi_ref, dinv_ref, w_ref,
                 olo_ref, ohi_ref=None, *, dout):
  dinv = dinv_ref[...]
  alo = jnp.maximum((slo_ref[...] + glo_ref[...]) * dinv, 0.0)
  ahi = jnp.maximum((shi_ref[...] + ghi_ref[...]) * dinv, 0.0)
  h = (jnp.dot(alo, w_ref[0:128, :], preferred_element_type=jnp.float32)
       + jnp.dot(ahi, w_ref[128:256, :], preferred_element_type=jnp.float32))
  g = h * dinv
  if dout == 256:
    olo_ref[...] = g[:, :128]
    ohi_ref[...] = g[:, 128:]
  else:
    olo_ref[...] = jnp.concatenate(
        [g, jnp.zeros((g.shape[0], 128 - dout), jnp.float32)], axis=1)


def _tc_mid(slo, shi, glo, ghi, dinv, w):
  dout = w.shape[1]
  grid = (N_NODES // _BN,)
  if dout == 256:
    out_specs = [pl.BlockSpec((_BN, 128), lambda i: (i, 0)),
                 pl.BlockSpec((_BN, 128), lambda i: (i, 0))]
    out_shape = [jax.ShapeDtypeStruct((N_NODES, 128), jnp.float32),
                 jax.ShapeDtypeStruct((N_NODES, 128), jnp.float32)]
  else:
    out_specs = [pl.BlockSpec((_BN, 128), lambda i: (i, 0))]
    out_shape = [jax.ShapeDtypeStruct((N_NODES, 128), jnp.float32)]
  return pl.pallas_call(
      functools.partial(_tc_mid_body, dout=dout),
      grid=grid,
      in_specs=[
          pl.BlockSpec((_BN, 128), lambda i: (i, 0)),
          pl.BlockSpec((_BN, 128), lambda i: (i, 0)),
          pl.BlockSpec((_BN, 128), lambda i: (i, 0)),
          pl.BlockSpec((_BN, 128), lambda i: (i, 0)),
          pl.BlockSpec((_BN, 1), lambda i: (i, 0)),
          pl.BlockSpec((D_FEAT, dout), lambda i: (0, 0)),
      ],
      out_specs=out_specs,
      out_shape=out_shape,
  )(slo, shi, glo, ghi, dinv, w)


def _tc_final_body(sp_ref, g_ref, dinv_ref, out_ref):
  z = sp_ref[0] + sp_ref[1] + g_ref[...]
  out_ref[...] = jnp.maximum(z * dinv_ref[...], 0.0)[:, :N_CLASSES]


def _tc_final(sp, g, dinv):
  grid = (N_NODES // _BN,)
  return pl.pallas_call(
      _tc_final_body,
      grid=grid,
      in_specs=[
          pl.BlockSpec((NC, _BN, 128), lambda i: (0, i, 0)),
          pl.BlockSpec((_BN, 128), lambda i: (i, 0)),
          pl.BlockSpec((_BN, 1), lambda i: (i, 0)),
      ],
      out_specs=pl.BlockSpec((_BN, N_CLASSES), lambda i: (i, 0)),
      out_shape=jax.ShapeDtypeStruct((N_NODES, N_CLASSES), jnp.float32),
  )(sp, g, dinv)


def kernel(x, edge_index, W0, W1, W2):
  src = edge_index[0].astype(jnp.int32)
  dst = edge_index[1].astype(jnp.int32)
  def pad_split(v, trash, groups):
    per = N_EDGES // groups
    per_pad = PH * CPP * CH * NS // groups
    v2 = v.reshape(groups, per)
    lanes = jnp.arange(per_pad - per, dtype=jnp.int32)
    gid = jnp.arange(groups, dtype=jnp.int32)
    if trash:
      fill = N_NODES + (lanes[None, :] + 8 * gid[:, None]) % N_TRASH
    else:
      fill = jnp.broadcast_to(lanes % CH, (groups, per_pad - per))
    return jnp.concatenate([v2, fill.astype(jnp.int32)], axis=1)

  src_f = pad_split(src, False, NS).reshape(NS, PH, CPP, CH)
  dst_f = pad_split(dst, True, NS).reshape(NS, PH, CPP, CH)
  src_e = pad_split(src, False, NC * NS).reshape(NC, NS, CPP, CH)
  dst_e = pad_split(dst, True, NC * NS).reshape(NC, NS, CPP, CH)

  degp = _deg_kernel(dst_e)
  glo0, ghi0, dinv = _tc1(x, W0, degp)
  slo0, shi0 = _edge128(glo0, ghi0, src_f, dst_f)
  glo1, ghi1 = _tc_mid(slo0, shi0, glo0, ghi0, dinv, W1)
  slo1, shi1 = _edge128(glo1, ghi1, src_f, dst_f)
  (g2,) = _tc_mid(slo1, shi1, glo1, ghi1, dinv, W2)
  s2p = _edge_partial(g2, src_e, dst_e)
  return _tc_final(s2p, g2, dinv)


# speedup vs baseline: 2.4228x; 1.0330x over previous
import functools

import jax
import jax.numpy as jnp
from jax import lax
from jax.experimental import pallas as pl
from jax.experimental.pallas import tpu as pltpu
from jax.experimental.pallas import tpu_sc as plsc

N_NODES = 10000
N_EDGES = 160000
D_FEAT = 256
N_CLASSES = 40

NC = 2
NS = 16
CH = 128
CPP = 40
PH = 2
N_TRASH = 128
ACC_ROWS = N_NODES + N_TRASH
NBUF = 2
ROWS_PER_TILE = 624
ZR = 16

_mesh = plsc.VectorSubcoreMesh(core_axis_name="c", subcore_axis_name="s")


def _fill_zero(ref, rows, width):
  zcols = width // 16
  def row(i, _):
    for k in range(zcols):
      ref[i, pl.ds(k * 16, 16)] = jnp.zeros((16,), jnp.float32)
    return 0
  lax.fori_loop(0, rows, row, 0)


def _zero_my_rows(acc_sp, zrow_v, s, sem):
  base = s * ROWS_PER_TILE
  total = jnp.where(s == NS - 1,
                    (ACC_ROWS - (NS - 1) * ROWS_PER_TILE) // ZR,
                    ROWS_PER_TILE // ZR)
  def fire(k, _):
    pltpu.make_async_copy(
        zrow_v, acc_sp.at[pl.ds(base + k * ZR, ZR)], sem).start()
    return 0
  def drain(k, _):
    pltpu.make_async_copy(
        zrow_v, acc_sp.at[pl.ds(base + k * ZR, ZR)], sem).wait()
    return 0
  lax.fori_loop(0, total, fire, 0)
  lax.fori_loop(0, total, drain, 0)


def _copy_my_rows(acc_sp, out_hbm, s):
  base = s * ROWS_PER_TILE
  pltpu.sync_copy(acc_sp.at[pl.ds(base, ROWS_PER_TILE)],
                  out_hbm.at[pl.ds(base, ROWS_PER_TILE)])

  @pl.when(s == NS - 1)
  def _():
    tail = NS * ROWS_PER_TILE
    pltpu.sync_copy(acc_sp.at[pl.ds(tail, N_NODES - tail)],
                    out_hbm.at[pl.ds(tail, N_NODES - tail)])


@functools.partial(
    pl.kernel,
    out_type=jax.ShapeDtypeStruct((NC, N_NODES, 16), jnp.float32),
    mesh=_mesh,
    scratch_types=[
        pltpu.VMEM((CPP, CH), jnp.int32),
        pltpu.VMEM((CH, 16), jnp.float32),
        pltpu.VMEM((ZR, 16), jnp.float32),
        pltpu.VMEM_SHARED((ACC_ROWS, 16), jnp.float32),
        pltpu.SemaphoreType.DMA,
    ],
)
def _deg_kernel(dst_hbm, degp_hbm, dstv, ones_v, zrow_v, deg_sp, sem):
  c = lax.axis_index("c")
  s = lax.axis_index("s")
  pltpu.sync_copy(dst_hbm.at[c, s], dstv)

  def fill_ones(i, _):
    ones_v[i, :] = jnp.ones((16,), jnp.float32)
    return 0
  lax.fori_loop(0, CH, fill_ones, 0)
  _fill_zero(zrow_v, ZR, 16)
  _zero_my_rows(deg_sp, zrow_v, s, sem)
  plsc.subcore_barrier()

  def chunk(j, _):
    pltpu.make_async_copy(ones_v, deg_sp.at[dstv.at[j]], sem).start(add=True)
    return 0
  lax.fori_loop(0, CPP, chunk, 0)
  def drain(j, _):
    pltpu.make_async_copy(ones_v, deg_sp.at[dstv.at[j]], sem).wait()
    return 0
  lax.fori_loop(0, CPP, drain, 0)
  plsc.subcore_barrier()

  _copy_my_rows(deg_sp, degp_hbm.at[c], s)


def _prime_gathers(g_hbm, srcv, rows_v, sem_g):
  for b in range(NBUF - 1):
    pltpu.make_async_copy(g_hbm.at[srcv.at[b]], rows_v.at[b], sem_g).start()


def _pipelined_edges(g_hbm, srcv, dstv, rows_v, acc_sp, sem_g, sem_s, nchunk,
                     prime=True):
  if prime:
    _prime_gathers(g_hbm, srcv, rows_v, sem_g)

  def chunk(j, _):
    cur = lax.rem(j, NBUF)

    @pl.when(j >= 1)
    def _():
      prev = lax.rem(j - 1, NBUF)
      pltpu.make_async_copy(
          rows_v.at[prev], acc_sp.at[dstv.at[j - 1]], sem_s).wait()

    @pl.when(j + NBUF - 1 < nchunk)
    def _():
      nxt = lax.rem(j + NBUF - 1, NBUF)
      pltpu.make_async_copy(
          g_hbm.at[srcv.at[j + NBUF - 1]], rows_v.at[nxt], sem_g).start()

    pltpu.make_async_copy(g_hbm.at[srcv.at[j]], rows_v.at[cur], sem_g).wait()
    pltpu.make_async_copy(
        rows_v.at[cur], acc_sp.at[dstv.at[j]], sem_s).start(add=True)
    return 0

  lax.fori_loop(0, nchunk, chunk, 0)
  last = (nchunk - 1) % NBUF
  pltpu.make_async_copy(
      rows_v.at[last], acc_sp.at[dstv.at[nchunk - 1]], sem_s).wait()


def _make_edge_kernel(feat):
  out_nd = jax.ShapeDtypeStruct((N_NODES, feat), jnp.float32)

  @functools.partial(
      pl.kernel,
      out_type=(out_nd, out_nd),
      mesh=_mesh,
      scratch_types=[
          pltpu.VMEM((CPP, CH), jnp.int32),
          pltpu.VMEM((CPP, CH), jnp.int32),
          pltpu.VMEM((NBUF, CH, feat), jnp.float32),
          pltpu.VMEM((ZR, feat), jnp.float32),
          pltpu.VMEM_SHARED((ACC_ROWS, feat), jnp.float32),
          pltpu.SemaphoreType.DMA,
          pltpu.SemaphoreType.DMA,
      ],
  )
  def edge_kernel(glo_hbm, ghi_hbm, src_hbm, dst_hbm, slo_hbm, shi_hbm,
                  srcv, dstv, rows_v, zrow_v, acc_sp, sem_g, sem_s):
    c = lax.axis_index("c")
    s = lax.axis_index("s")

    _fill_zero(zrow_v, ZR, feat)

    def run(g_hbm):
      pltpu.sync_copy(src_hbm.at[s, 0], srcv)
      pltpu.sync_copy(dst_hbm.at[s, 0], dstv)
      _prime_gathers(g_hbm, srcv, rows_v, sem_g)
      _zero_my_rows(acc_sp, zrow_v, s, sem_s)
      plsc.subcore_barrier()
      _pipelined_edges(g_hbm, srcv, dstv, rows_v, acc_sp, sem_g, sem_s, CPP,
                       prime=False)
      pltpu.sync_copy(src_hbm.at[s, 1], srcv)
      pltpu.sync_copy(dst_hbm.at[s, 1], dstv)
      _pipelined_edges(g_hbm, srcv, dstv, rows_v, acc_sp, sem_g, sem_s, CPP)

    @pl.when(c == 0)
    def _():
      run(glo_hbm)

    @pl.when(c == 1)
    def _():
      run(ghi_hbm)

    plsc.subcore_barrier()

    @pl.when(c == 0)
    def _():
      _copy_my_rows(acc_sp, slo_hbm, s)

    @pl.when(c == 1)
    def _():
      _copy_my_rows(acc_sp, shi_hbm, s)

  return edge_kernel


_edge128 = _make_edge_kernel(128)


@functools.partial(
    pl.kernel,
    out_type=jax.ShapeDtypeStruct((NC, N_NODES, 128), jnp.float32),
    mesh=_mesh,
    scratch_types=[
        pltpu.VMEM((CPP, CH), jnp.int32),
        pltpu.VMEM((CPP, CH), jnp.int32),
        pltpu.VMEM((NBUF, CH, 128), jnp.float32),
        pltpu.VMEM((ZR, 128), jnp.float32),
        pltpu.VMEM_SHARED((ACC_ROWS, 128), jnp.float32),
        pltpu.SemaphoreType.DMA,
        pltpu.SemaphoreType.DMA,
    ],
)
def _edge_partial(g_hbm, src_hbm, dst_hbm, sp_hbm,
                  srcv, dstv, rows_v, zrow_v, acc_sp, sem_g, sem_s):
  c = lax.axis_index("c")
  s = lax.axis_index("s")
  pltpu.sync_copy(src_hbm.at[c, s], srcv)
  pltpu.sync_copy(dst_hbm.at[c, s], dstv)

  _fill_zero(zrow_v, ZR, 128)
  _prime_gathers(g_hbm, srcv, rows_v, sem_g)
  _zero_my_rows(acc_sp, zrow_v, s, sem_s)
  plsc.subcore_barrier()

  _pipelined_edges(g_hbm, srcv, dstv, rows_v, acc_sp, sem_g, sem_s, CPP,
                   prime=False)
  plsc.subcore_barrier()

  _copy_my_rows(acc_sp, sp_hbm.at[c], s)


_BN = 1000


def _tc1_body(x_ref, w_ref, degp_ref, glo_ref, ghi_ref, dinv_ref):
  deg = 1.0 + degp_ref[0, :, 0:1] + degp_ref[1, :, 0:1]
  dinv = lax.rsqrt(deg)
  h = jnp.dot(x_ref[...], w_ref[...], preferred_element_type=jnp.float32)
  g = h * dinv
  glo_ref[...] = g[:, :128]
  ghi_ref[...] = g[:, 128:]
  dinv_ref[...] = dinv


def _tc1(x, w0, degp):
  grid = (N_NODES // _BN,)
  return pl.pallas_call(
      _tc1_body,
      grid=grid,
      in_specs=[
          pl.BlockSpec((_BN, D_FEAT), lambda i: (i, 0)),
          pl.BlockSpec((D_FEAT, D_FEAT), lambda i: (0, 0)),
          pl.BlockSpec((NC, _BN, 16), lambda i: (0, i, 0)),
      ],
      out_specs=[
          pl.BlockSpec((_BN, 128), lambda i: (i, 0)),
          pl.BlockSpec((_BN, 128), lambda i: (i, 0)),
          pl.BlockSpec((_BN, 1), lambda i: (i, 0)),
      ],
      out_shape=[
          jax.ShapeDtypeStruct((N_NODES, 128), jnp.float32),
          jax.ShapeDtypeStruct((N_NODES, 128), jnp.float32),
          jax.ShapeDtypeStruct((N_NODES, 1), jnp.float32),
      ],
  )(x, w0, degp)


def _tc_mid_body(slo_ref, shi_ref, glo_ref, ghi_ref, dinv_ref, w_ref,
                 olo_ref, ohi_ref=None, *, dout):
  dinv = dinv_ref[...]
  alo = jnp.maximum((slo_ref[...] + glo_ref[...]) * dinv, 0.0)
  ahi = jnp.maximum((shi_ref[...] + ghi_ref[...]) * dinv, 0.0)
  h = (jnp.dot(alo, w_ref[0:128, :], preferred_element_type=jnp.float32)
       + jnp.dot(ahi, w_ref[128:256, :], preferred_element_type=jnp.float32))
  g = h * dinv
  if dout == 256:
    olo_ref[...] = g[:, :128]
    ohi_ref[...] = g[:, 128:]
  else:
    olo_ref[...] = jnp.concatenate(
        [g, jnp.zeros((g.shape[0], 128 - dout), jnp.float32)], axis=1)


def _tc_mid(slo, shi, glo, ghi, dinv, w):
  dout = w.shape[1]
  grid = (N_NODES // _BN,)
  if dout == 256:
    out_specs = [pl.BlockSpec((_BN, 128), lambda i: (i, 0)),
                 pl.BlockSpec((_BN, 128), lambda i: (i, 0))]
    out_shape = [jax.ShapeDtypeStruct((N_NODES, 128), jnp.float32),
                 jax.ShapeDtypeStruct((N_NODES, 128), jnp.float32)]
  else:
    out_specs = [pl.BlockSpec((_BN, 128), lambda i: (i, 0))]
    out_shape = [jax.ShapeDtypeStruct((N_NODES, 128), jnp.float32)]
  return pl.pallas_call(
      functools.partial(_tc_mid_body, dout=dout),
      grid=grid,
      in_specs=[
          pl.BlockSpec((_BN, 128), lambda i: (i, 0)),
          pl.BlockSpec((_BN, 128), lambda i: (i, 0)),
          pl.BlockSpec((_BN, 128), lambda i: (i, 0)),
          pl.BlockSpec((_BN, 128), lambda i: (i, 0)),
          pl.BlockSpec((_BN, 1), lambda i: (i, 0)),
          pl.BlockSpec((D_FEAT, dout), lambda i: (0, 0)),
      ],
      out_specs=out_specs,
      out_shape=out_shape,
  )(slo, shi, glo, ghi, dinv, w)


def _tc_final_body(sp_ref, g_ref, dinv_ref, out_ref):
  z = sp_ref[0] + sp_ref[1] + g_ref[...]
  out_ref[...] = jnp.maximum(z * dinv_ref[...], 0.0)[:, :N_CLASSES]


def _tc_final(sp, g, dinv):
  grid = (N_NODES // _BN,)
  return pl.pallas_call(
      _tc_final_body,
      grid=grid,
      in_specs=[
          pl.BlockSpec((NC, _BN, 128), lambda i: (0, i, 0)),
          pl.BlockSpec((_BN, 128), lambda i: (i, 0)),
          pl.BlockSpec((_BN, 1), lambda i: (i, 0)),
      ],
      out_specs=pl.BlockSpec((_BN, N_CLASSES), lambda i: (i, 0)),
      out_shape=jax.ShapeDtypeStruct((N_NODES, N_CLASSES), jnp.float32),
  )(sp, g, dinv)


def kernel(x, edge_index, W0, W1, W2):
  src = edge_index[0].astype(jnp.int32)
  dst = edge_index[1].astype(jnp.int32)
  def pad_split(v, trash, groups):
    per = N_EDGES // groups
    per_pad = PH * CPP * CH * NS // groups
    v2 = v.reshape(groups, per)
    lanes = jnp.arange(per_pad - per, dtype=jnp.int32)
    gid = jnp.arange(groups, dtype=jnp.int32)
    if trash:
      fill = N_NODES + (lanes[None, :] + 8 * gid[:, None]) % N_TRASH
    else:
      fill = jnp.broadcast_to(lanes % CH, (groups, per_pad - per))
    return jnp.concatenate([v2, fill.astype(jnp.int32)], axis=1)

  src_f = pad_split(src, False, NS).reshape(NS, PH, CPP, CH)
  dst_f = pad_split(dst, True, NS).reshape(NS, PH, CPP, CH)
  src_e = pad_split(src, False, NC * NS).reshape(NC, NS, CPP, CH)
  dst_e = pad_split(dst, True, NC * NS).reshape(NC, NS, CPP, CH)

  degp = _deg_kernel(dst_e)
  glo0, ghi0, dinv = _tc1(x, W0, degp)
  slo0, shi0 = _edge128(glo0, ghi0, src_f, dst_f)
  glo1, ghi1 = _tc_mid(slo0, shi0, glo0, ghi0, dinv, W1)
  slo1, shi1 = _edge128(glo1, ghi1, src_f, dst_f)
  (g2,) = _tc_mid(slo1, shi1, glo1, ghi1, dinv, W2)
  s2p = _edge_partial(g2, src_e, dst_e)
  return _tc_final(s2p, g2, dinv)
